# Initial kernel scaffold; baseline (speedup 1.0000x reference)
import functools

import jax
import jax.numpy as jnp
from jax import lax
from jax.experimental import pallas as pl
from jax.experimental.pallas import tpu as pltpu
from jax.experimental.pallas import tpu_sc as plsc

N = 10000
E = 320000
D = 128
G = 128
D_OUT = 64
EPS_BN = 1e-5

NC = 2
NS = 16
NW = NC * NS
EPW = E // NW
K = 80
NCHUNK = EPW // K
NP = 10240
RPT = N // NS
RCP = 125


def _sds(shape, dtype=jnp.float32):
    return jax.ShapeDtypeStruct(shape, dtype)


_mesh = plsc.VectorSubcoreMesh(
    core_axis_name="c", subcore_axis_name="s", num_cores=NC, num_subcores=NS
)


@functools.partial(
    pl.kernel,
    out_type=[_sds((NC, N, D)), _sds((NW, NP))],
    mesh=_mesh,
    scratch_types=[
        pltpu.VMEM((N,), jnp.float32),
        pltpu.VMEM((N,), jnp.float32),
        pltpu.VMEM((NP,), jnp.float32),
        pltpu.VMEM((EPW,), jnp.int32),
        pltpu.VMEM((EPW,), jnp.int32),
        pltpu.VMEM((K,), jnp.int32),
        pltpu.VMEM((K,), jnp.int32),
        pltpu.VMEM((K,), jnp.float32),
        pltpu.VMEM((K, D), jnp.float32),
        pltpu.VMEM((RCP, D), jnp.float32),
        pltpu.VMEM_SHARED((N, D), jnp.float32),
        pltpu.SemaphoreType.DMA,
    ],
)
def _edge_phase(edge_hbm, asrc_hbm, adst_hbm, h_hbm, acc_out, den_out,
                asrc_v, adst_v, den_v, sall_v, dall_v, sidx_v, didx_v,
                ex_v, rows_v, stage_v, acc_sp, sem):
    c = lax.axis_index("c")
    s = lax.axis_index("s")
    wid = c * NS + s
    ebase = wid * EPW

    pltpu.sync_copy(asrc_hbm, asrc_v)
    pltpu.sync_copy(adst_hbm, adst_v)
    pltpu.sync_copy(edge_hbm.at[0, pl.ds(ebase, EPW)], sall_v)
    pltpu.sync_copy(edge_hbm.at[1, pl.ds(ebase, EPW)], dall_v)

    zeros16 = jnp.zeros((16,), jnp.float32)

    @pl.loop(0, NP // 16)
    def _zero_den(g):
        den_v[pl.ds(g * 16, 16)] = zeros16

    @pl.loop(0, RCP)
    def _zero_stage(k):
        for j in range(D // 16):
            stage_v[k, pl.ds(j * 16, 16)] = zeros16

    for i in range(RPT // RCP):
        pltpu.sync_copy(stage_v, acc_sp.at[pl.ds(s * RPT + i * RCP, RCP)])
    plsc.subcore_barrier()

    @pl.loop(0, NCHUNK)
    def _chunk(j):
        base = j * K
        pltpu.sync_copy(sall_v.at[pl.ds(base, K)], sidx_v)
        pltpu.sync_copy(dall_v.at[pl.ds(base, K)], didx_v)
        pltpu.async_copy(h_hbm.at[sidx_v], rows_v, sem).wait()
        for g in range(K // 16):
            s16 = sidx_v[pl.ds(g * 16, 16)]
            d16 = didx_v[pl.ds(g * 16, 16)]
            a = plsc.load_gather(asrc_v, [s16]) + plsc.load_gather(adst_v, [d16])
            ex = jnp.exp(jnp.maximum(a, 0.2 * a))
            plsc.addupdate_scatter(den_v, [d16], ex)
            ex_v[pl.ds(g * 16, 16)] = ex

        @pl.loop(0, K)
        def _scale(k):
            e16 = plsc.load_gather(ex_v, [jnp.full((16,), k, jnp.int32)])
            for j in range(D // 16):
                sl = pl.ds(j * 16, 16)
                rows_v[k, sl] = rows_v[k, sl] * e16

        pltpu.sync_copy(rows_v, acc_sp.at[didx_v], add=True)

    plsc.subcore_barrier()
    pltpu.sync_copy(den_v, den_out.at[wid])
    for i in range(RPT // RCP):
        r0 = s * RPT + i * RCP
        pltpu.sync_copy(acc_sp.at[pl.ds(r0, RCP)], stage_v)
        pltpu.sync_copy(stage_v, acc_out.at[c, pl.ds(r0, RCP)])


def _tc1_body(x_ref, w_ref, as_ref, ad_ref, h_ref, av_ref, bv_ref):
    h = jnp.dot(x_ref[...], w_ref[...], preferred_element_type=jnp.float32)
    h_ref[...] = h
    av_ref[...] = jnp.sum(h * as_ref[...], axis=1, keepdims=True)
    bv_ref[...] = jnp.sum(h * ad_ref[...], axis=1, keepdims=True)


def _den_col(den_ref):
    ones = jnp.ones((NW, 1), jnp.float32)
    col = lax.dot_general(den_ref[...], ones, (((0,), (0,)), ((), ())),
                          preferred_element_type=jnp.float32)
    return col[:N]


def _bn_relu(y, g_ref, be_ref):
    mean = jnp.mean(y, axis=0, keepdims=True)
    var = jnp.mean((y - mean) ** 2, axis=0, keepdims=True)
    return jnp.maximum(
        g_ref[...] * (y - mean) / jnp.sqrt(var + EPS_BN) + be_ref[...], 0.0)


def _tc2_body(acc_ref, den_ref, b_ref, g_ref, be_ref, w2_ref, as2_ref,
              ad2_ref, h2_ref, av_ref, bv_ref):
    y = (acc_ref[0] + acc_ref[1]) / (_den_col(den_ref) + 1e-16) + b_ref[...]
    z = _bn_relu(y, g_ref, be_ref)
    h2 = jnp.dot(z, w2_ref[...], preferred_element_type=jnp.float32)
    h2_ref[...] = h2
    av_ref[...] = jnp.sum(h2 * as2_ref[...], axis=1, keepdims=True)
    bv_ref[...] = jnp.sum(h2 * ad2_ref[...], axis=1, keepdims=True)


def _tc3_body(acc_ref, den_ref, batch_ref, b_ref, g_ref, be_ref, wfc_ref,
              bfc_ref, out_ref):
    y = (acc_ref[0] + acc_ref[1]) / (_den_col(den_ref) + 1e-16) + b_ref[...]
    z = _bn_relu(y, g_ref, be_ref)
    onehot = (batch_ref[...] == lax.broadcasted_iota(jnp.int32, (1, G), 1)
              ).astype(jnp.float32)
    sums = lax.dot_general(onehot, z, (((0,), (0,)), ((), ())),
                           preferred_element_type=jnp.float32)
    cnt = lax.dot_general(onehot, jnp.ones((N, 1), jnp.float32),
                          (((0,), (0,)), ((), ())),
                          preferred_element_type=jnp.float32)
    pool = sums / jnp.maximum(cnt, 1.0)
    out_ref[...] = jnp.dot(pool, wfc_ref[...],
                           preferred_element_type=jnp.float32) + bfc_ref[...]


_tc1 = pl.pallas_call(
    _tc1_body, out_shape=[_sds((N, D)), _sds((N, 1)), _sds((N, 1))])
_tc2 = pl.pallas_call(
    _tc2_body, out_shape=[_sds((N, D)), _sds((N, 1)), _sds((N, 1))])
_tc3 = pl.pallas_call(_tc3_body, out_shape=_sds((G, D_OUT)))


def kernel(x, edge_index, batch, W1, att_src1, att_dst1, b1, gamma1, beta1,
           W2, att_src2, att_dst2, b2, gamma2, beta2, Wfc, bfc):
    h1, a1s, a1d = _tc1(x, W1, att_src1, att_dst1)
    acc1, den1 = _edge_phase(edge_index, a1s.reshape(N), a1d.reshape(N), h1)
    h2, a2s, a2d = _tc2(acc1, den1, b1.reshape(1, D), gamma1.reshape(1, D),
                        beta1.reshape(1, D), W2, att_src2, att_dst2)
    acc2, den2 = _edge_phase(edge_index, a2s.reshape(N), a2d.reshape(N), h2)
    return _tc3(acc2, den2, batch.reshape(N, 1), b2.reshape(1, D),
                gamma2.reshape(1, D), beta2.reshape(1, D), Wfc,
                bfc.reshape(1, D_OUT))


# speedup vs baseline: 21.2638x; 21.2638x over previous
import functools

import jax
import jax.numpy as jnp
from jax import lax
from jax.experimental import pallas as pl
from jax.experimental.pallas import tpu as pltpu
from jax.experimental.pallas import tpu_sc as plsc

N = 10000
E = 320000
D = 128
G = 128
D_OUT = 64
EPS_BN = 1e-5

NC = 2
NS = 16
NW = NC * NS
EPW = E // NW
K = 80
NCHUNK = EPW // K
NP = 10240
RPT = NP // NS
RCP = K


def _sds(shape, dtype=jnp.float32):
    return jax.ShapeDtypeStruct(shape, dtype)


_mesh = plsc.VectorSubcoreMesh(
    core_axis_name="c", subcore_axis_name="s", num_cores=NC, num_subcores=NS
)


@functools.partial(
    pl.kernel,
    out_type=[_sds((NC, NP, D)), _sds((NW, NP))],
    mesh=_mesh,
    compiler_params=pltpu.CompilerParams(needs_layout_passes=False),
    scratch_types=[
        pltpu.VMEM((N,), jnp.float32),
        pltpu.VMEM((N,), jnp.float32),
        pltpu.VMEM((NP,), jnp.float32),
        pltpu.VMEM((K,), jnp.int32),
        pltpu.VMEM((K,), jnp.int32),
        pltpu.VMEM((K,), jnp.float32),
        pltpu.VMEM((K, D), jnp.float32),
        pltpu.VMEM_SHARED((NP, D), jnp.float32),
        pltpu.SemaphoreType.DMA,
    ],
)
def _edge_phase(src_hbm, dst_hbm, asrc_hbm, adst_hbm, h_hbm, acc_out, den_out,
                asrc_v, adst_v, den_v, sidx_v, didx_v,
                ex_v, rows_v, acc_sp, sem):
    c = lax.axis_index("c")
    s = lax.axis_index("s")
    wid = c * NS + s
    ebase = wid * EPW

    pltpu.sync_copy(asrc_hbm, asrc_v)
    pltpu.sync_copy(adst_hbm, adst_v)

    zeros16 = jnp.zeros((16,), jnp.float32)

    @pl.loop(0, NP // 16)
    def _zero_den(g):
        den_v[pl.ds(g * 16, 16)] = zeros16

    @pl.loop(0, RCP)
    def _zero_stage(k):
        for j in range(D // 16):
            rows_v[k, pl.ds(j * 16, 16)] = zeros16

    for i in range(RPT // RCP):
        pltpu.sync_copy(rows_v, acc_sp.at[pl.ds(s * RPT + i * RCP, RCP)])
    plsc.subcore_barrier()

    @pl.loop(0, NCHUNK)
    def _chunk(j):
        base = ebase + j * K
        pltpu.sync_copy(src_hbm.at[pl.ds(base, K)], sidx_v)
        pltpu.sync_copy(dst_hbm.at[pl.ds(base, K)], didx_v)
        pltpu.async_copy(h_hbm.at[sidx_v], rows_v, sem).wait()
        for g in range(K // 16):
            s16 = sidx_v[pl.ds(g * 16, 16)]
            d16 = didx_v[pl.ds(g * 16, 16)]
            a = plsc.load_gather(asrc_v, [s16]) + plsc.load_gather(adst_v, [d16])
            ex = jnp.exp(jnp.maximum(a, 0.2 * a))
            plsc.addupdate_scatter(den_v, [d16], ex)
            ex_v[pl.ds(g * 16, 16)] = ex

        @pl.loop(0, K)
        def _scale(k):
            e16 = plsc.load_gather(ex_v, [jnp.full((16,), k, jnp.int32)])
            for j in range(D // 16):
                sl = pl.ds(j * 16, 16)
                rows_v[k, sl] = rows_v[k, sl] * e16

        pltpu.sync_copy(rows_v, acc_sp.at[didx_v], add=True)

    plsc.subcore_barrier()
    pltpu.sync_copy(den_v, den_out.at[wid])
    for i in range(RPT // RCP):
        r0 = s * RPT + i * RCP
        pltpu.sync_copy(acc_sp.at[pl.ds(r0, RCP)], rows_v)
        pltpu.sync_copy(rows_v, acc_out.at[c, pl.ds(r0, RCP)])


def _tc1_body(x_ref, w_ref, as_ref, ad_ref, h_ref, av_ref, bv_ref):
    h = jnp.dot(x_ref[...], w_ref[...], preferred_element_type=jnp.float32)
    h_ref[...] = h
    av_ref[...] = jnp.sum(h * as_ref[...], axis=1, keepdims=True)
    bv_ref[...] = jnp.sum(h * ad_ref[...], axis=1, keepdims=True)


def _den_col(den_ref):
    ones = jnp.ones((NW, 1), jnp.float32)
    col = lax.dot_general(den_ref[...], ones, (((0,), (0,)), ((), ())),
                          preferred_element_type=jnp.float32)
    return col[:N]


def _bn_relu(y, g_ref, be_ref):
    mean = jnp.mean(y, axis=0, keepdims=True)
    var = jnp.mean((y - mean) ** 2, axis=0, keepdims=True)
    return jnp.maximum(
        g_ref[...] * (y - mean) / jnp.sqrt(var + EPS_BN) + be_ref[...], 0.0)


def _tc2_body(acc_ref, den_ref, b_ref, g_ref, be_ref, w2_ref, as2_ref,
              ad2_ref, h2_ref, av_ref, bv_ref):
    a = acc_ref[...]
    y = (a[0, :N] + a[1, :N]) / (_den_col(den_ref) + 1e-16) + b_ref[...]
    z = _bn_relu(y, g_ref, be_ref)
    h2 = jnp.dot(z, w2_ref[...], preferred_element_type=jnp.float32)
    h2_ref[...] = h2
    av_ref[...] = jnp.sum(h2 * as2_ref[...], axis=1, keepdims=True)
    bv_ref[...] = jnp.sum(h2 * ad2_ref[...], axis=1, keepdims=True)


def _tc3_body(acc_ref, den_ref, batch_ref, b_ref, g_ref, be_ref, wfc_ref,
              bfc_ref, out_ref):
    a = acc_ref[...]
    y = (a[0, :N] + a[1, :N]) / (_den_col(den_ref) + 1e-16) + b_ref[...]
    z = _bn_relu(y, g_ref, be_ref)
    onehot = (batch_ref[...] == lax.broadcasted_iota(jnp.int32, (1, G), 1)
              ).astype(jnp.float32)
    sums = lax.dot_general(onehot, z, (((0,), (0,)), ((), ())),
                           preferred_element_type=jnp.float32)
    cnt = lax.dot_general(onehot, jnp.ones((N, 1), jnp.float32),
                          (((0,), (0,)), ((), ())),
                          preferred_element_type=jnp.float32)
    pool = sums / jnp.maximum(cnt, 1.0)
    out_ref[...] = jnp.dot(pool, wfc_ref[...],
                           preferred_element_type=jnp.float32) + bfc_ref[...]


_tc1 = pl.pallas_call(
    _tc1_body, out_shape=[_sds((N, D)), _sds((N, 1)), _sds((N, 1))])
_tc2 = pl.pallas_call(
    _tc2_body, out_shape=[_sds((N, D)), _sds((N, 1)), _sds((N, 1))])
_tc3 = pl.pallas_call(_tc3_body, out_shape=_sds((G, D_OUT)))


def kernel(x, edge_index, batch, W1, att_src1, att_dst1, b1, gamma1, beta1,
           W2, att_src2, att_dst2, b2, gamma2, beta2, Wfc, bfc):
    src = edge_index[0]
    dst = edge_index[1]
    h1, a1s, a1d = _tc1(x, W1, att_src1, att_dst1)
    acc1, den1 = _edge_phase(src, dst, a1s.reshape(N), a1d.reshape(N), h1)
    h2, a2s, a2d = _tc2(acc1, den1, b1.reshape(1, D), gamma1.reshape(1, D),
                        beta1.reshape(1, D), W2, att_src2, att_dst2)
    acc2, den2 = _edge_phase(src, dst, a2s.reshape(N), a2d.reshape(N), h2)
    return _tc3(acc2, den2, batch.reshape(N, 1), b2.reshape(1, D),
                gamma2.reshape(1, D), beta2.reshape(1, D), Wfc,
                bfc.reshape(1, D_OUT))


# speedup vs baseline: 43.9195x; 2.0655x over previous
import functools

import jax
import jax.numpy as jnp
from jax import lax
from jax.experimental import pallas as pl
from jax.experimental.pallas import tpu as pltpu
from jax.experimental.pallas import tpu_sc as plsc

N = 10000
E = 320000
D = 128
G = 128
D_OUT = 64
EPS_BN = 1e-5

NC = 2
NS = 16
NW = NC * NS
EPW = E // NW
K = 80
NCHUNK = EPW // K
NP = 10240
RPT = NP // NS
RCP = K


def _sds(shape, dtype=jnp.float32):
    return jax.ShapeDtypeStruct(shape, dtype)


_mesh = plsc.VectorSubcoreMesh(
    core_axis_name="c", subcore_axis_name="s", num_cores=NC, num_subcores=NS
)


@functools.partial(
    pl.kernel,
    out_type=[_sds((NC, NP, D)), _sds((NW, NP))],
    mesh=_mesh,
    compiler_params=pltpu.CompilerParams(needs_layout_passes=False),
    scratch_types=[
        pltpu.VMEM((NP,), jnp.float32),
        pltpu.VMEM((K,), jnp.float32),
        [pltpu.VMEM((K,), jnp.int32)] * 2,
        [pltpu.VMEM((K,), jnp.int32)] * 2,
        [pltpu.VMEM((K,), jnp.int32)] * 2,
        [pltpu.VMEM((K,), jnp.float32)] * 2,
        [pltpu.VMEM((K,), jnp.float32)] * 2,
        [pltpu.VMEM((K, D), jnp.float32)] * 2,
        [pltpu.SemaphoreType.DMA] * 2,
        [pltpu.SemaphoreType.DMA] * 2,
        [pltpu.SemaphoreType.DMA] * 2,
        [pltpu.SemaphoreType.DMA] * 2,
        [pltpu.SemaphoreType.DMA] * 2,
        pltpu.VMEM_SHARED((NP, D), jnp.float32),
    ],
)
def _edge_phase(src_hbm, dst_hbm, asrc_hbm, adst_hbm, h_hbm, acc_out, den_out,
                den_v, ex_v, sidx, didx, dsct, av, bv, rows,
                isem, gsem, asem, bsem, ssem, acc_sp):
    c = lax.axis_index("c")
    s = lax.axis_index("s")
    wid = c * NS + s
    ebase = wid * EPW

    zeros16 = jnp.zeros((16,), jnp.float32)

    @pl.loop(0, NP // 16)
    def _zero_den(g):
        den_v[pl.ds(g * 16, 16)] = zeros16

    @pl.loop(0, RCP)
    def _zero_stage(k):
        for j in range(D // 16):
            rows[0][k, pl.ds(j * 16, 16)] = zeros16

    for i in range(RPT // RCP):
        pltpu.sync_copy(rows[0], acc_sp.at[pl.ds(s * RPT + i * RCP, RCP)])
    plsc.subcore_barrier()

    def issue_idx(j, b):
        base = ebase + j * K
        pltpu.async_copy(src_hbm.at[pl.ds(base, K)], sidx[b], isem[b])
        pltpu.async_copy(dst_hbm.at[pl.ds(base, K)], didx[b], isem[b])

    def wait_idx(b):
        pltpu.make_async_copy(src_hbm.at[pl.ds(0, K)], sidx[b], isem[b]).wait()
        pltpu.make_async_copy(dst_hbm.at[pl.ds(0, K)], didx[b], isem[b]).wait()

    def issue_gathers(b):
        pltpu.async_copy(h_hbm.at[sidx[b]], rows[b], gsem[b])
        pltpu.async_copy(asrc_hbm.at[sidx[b]], av[b], asem[b])
        pltpu.async_copy(adst_hbm.at[didx[b]], bv[b], bsem[b])

    def wait_gathers(b):
        pltpu.make_async_copy(h_hbm.at[sidx[b]], rows[b], gsem[b]).wait()
        pltpu.make_async_copy(asrc_hbm.at[sidx[b]], av[b], asem[b]).wait()
        pltpu.make_async_copy(adst_hbm.at[didx[b]], bv[b], bsem[b]).wait()

    def wait_scatter(b):
        pltpu.make_async_copy(rows[b], acc_sp.at[dsct[b]], ssem[b]).wait()

    issue_idx(0, 0)
    wait_idx(0)
    issue_gathers(0)

    @pl.loop(0, NCHUNK + 1, step=2)
    def _chunks(j0):
        for u in range(2):
            j = j0 + u
            b = u
            n = 1 - u

            @pl.when(j < NCHUNK)
            def _():
                @pl.when(j + 1 < NCHUNK)
                def _():
                    issue_idx(j + 1, n)

                wait_gathers(b)
                for g in range(K // 16):
                    sl = pl.ds(g * 16, 16)
                    d16 = didx[b][sl]
                    a = av[b][sl] + bv[b][sl]
                    ex = jnp.exp(jnp.maximum(a, 0.2 * a))
                    plsc.addupdate_scatter(den_v, [d16], ex)
                    ex_v[sl] = ex
                    dsct[b][sl] = d16

                @pl.when(j > 0)
                def _():
                    wait_scatter(n)

                @pl.when(j + 1 < NCHUNK)
                def _():
                    wait_idx(n)
                    issue_gathers(n)

                @pl.loop(0, K)
                def _scale(k):
                    e16 = plsc.load_gather(ex_v, [jnp.full((16,), k, jnp.int32)])
                    for jj in range(D // 16):
                        sl = pl.ds(jj * 16, 16)
                        rows[b][k, sl] = rows[b][k, sl] * e16

                pltpu.async_copy(rows[b], acc_sp.at[dsct[b]], ssem[b],
                                 add=True)

    wait_scatter((NCHUNK - 1) % 2)

    plsc.subcore_barrier()
    pltpu.sync_copy(den_v, den_out.at[wid])
    for i in range(RPT // RCP):
        r0 = s * RPT + i * RCP
        pltpu.sync_copy(acc_sp.at[pl.ds(r0, RCP)], rows[0])
        pltpu.sync_copy(rows[0], acc_out.at[c, pl.ds(r0, RCP)])


def _tc1_body(x_ref, w_ref, as_ref, ad_ref, h_ref, av_ref, bv_ref):
    h = jnp.dot(x_ref[...], w_ref[...], preferred_element_type=jnp.float32)
    h_ref[...] = h
    av_ref[...] = jnp.sum(h * as_ref[...], axis=1, keepdims=True)
    bv_ref[...] = jnp.sum(h * ad_ref[...], axis=1, keepdims=True)


def _den_col(den_ref):
    ones = jnp.ones((NW, 1), jnp.float32)
    col = lax.dot_general(den_ref[...], ones, (((0,), (0,)), ((), ())),
                          preferred_element_type=jnp.float32)
    return col[:N]


def _bn_relu(y, g_ref, be_ref):
    mean = jnp.mean(y, axis=0, keepdims=True)
    var = jnp.mean((y - mean) ** 2, axis=0, keepdims=True)
    return jnp.maximum(
        g_ref[...] * (y - mean) / jnp.sqrt(var + EPS_BN) + be_ref[...], 0.0)


def _tc2_body(acc_ref, den_ref, b_ref, g_ref, be_ref, w2_ref, as2_ref,
              ad2_ref, h2_ref, av_ref, bv_ref):
    a = acc_ref[...]
    y = (a[0, :N] + a[1, :N]) / (_den_col(den_ref) + 1e-16) + b_ref[...]
    z = _bn_relu(y, g_ref, be_ref)
    h2 = jnp.dot(z, w2_ref[...], preferred_element_type=jnp.float32)
    h2_ref[...] = h2
    av_ref[...] = jnp.sum(h2 * as2_ref[...], axis=1, keepdims=True)
    bv_ref[...] = jnp.sum(h2 * ad2_ref[...], axis=1, keepdims=True)


def _tc3_body(acc_ref, den_ref, batch_ref, b_ref, g_ref, be_ref, wfc_ref,
              bfc_ref, out_ref):
    a = acc_ref[...]
    y = (a[0, :N] + a[1, :N]) / (_den_col(den_ref) + 1e-16) + b_ref[...]
    z = _bn_relu(y, g_ref, be_ref)
    onehot = (batch_ref[...] == lax.broadcasted_iota(jnp.int32, (1, G), 1)
              ).astype(jnp.float32)
    sums = lax.dot_general(onehot, z, (((0,), (0,)), ((), ())),
                           preferred_element_type=jnp.float32)
    cnt = lax.dot_general(onehot, jnp.ones((N, 1), jnp.float32),
                          (((0,), (0,)), ((), ())),
                          preferred_element_type=jnp.float32)
    pool = sums / jnp.maximum(cnt, 1.0)
    out_ref[...] = jnp.dot(pool, wfc_ref[...],
                           preferred_element_type=jnp.float32) + bfc_ref[...]


_tc1 = pl.pallas_call(
    _tc1_body, out_shape=[_sds((N, D)), _sds((N, 1)), _sds((N, 1))])
_tc2 = pl.pallas_call(
    _tc2_body, out_shape=[_sds((N, D)), _sds((N, 1)), _sds((N, 1))])
_tc3 = pl.pallas_call(_tc3_body, out_shape=_sds((G, D_OUT)))


def kernel(x, edge_index, batch, W1, att_src1, att_dst1, b1, gamma1, beta1,
           W2, att_src2, att_dst2, b2, gamma2, beta2, Wfc, bfc):
    src = edge_index[0]
    dst = edge_index[1]
    h1, a1s, a1d = _tc1(x, W1, att_src1, att_dst1)
    acc1, den1 = _edge_phase(src, dst, a1s.reshape(N), a1d.reshape(N), h1)
    h2, a2s, a2d = _tc2(acc1, den1, b1.reshape(1, D), gamma1.reshape(1, D),
                        beta1.reshape(1, D), W2, att_src2, att_dst2)
    acc2, den2 = _edge_phase(src, dst, a2s.reshape(N), a2d.reshape(N), h2)
    return _tc3(acc2, den2, batch.reshape(N, 1), b2.reshape(1, D),
                gamma2.reshape(1, D), beta2.reshape(1, D), Wfc,
                bfc.reshape(1, D_OUT))


# speedup vs baseline: 51.9541x; 1.1829x over previous
import functools

import jax
import jax.numpy as jnp
from jax import lax
from jax.experimental import pallas as pl
from jax.experimental.pallas import tpu as pltpu
from jax.experimental.pallas import tpu_sc as plsc

N = 10000
E = 320000
D = 128
G = 128
D_OUT = 64
EPS_BN = 1e-5

NC = 2
NS = 16
NW = NC * NS
EPW = E // NW
K = 80
NCHUNK = EPW // K
NP = 10240
RPT = NP // NS
RCP = K


def _sds(shape, dtype=jnp.float32):
    return jax.ShapeDtypeStruct(shape, dtype)


_mesh = plsc.VectorSubcoreMesh(
    core_axis_name="c", subcore_axis_name="s", num_cores=NC, num_subcores=NS
)


@functools.partial(
    pl.kernel,
    out_type=[_sds((NC, NP, D)), _sds((NW, NP))],
    mesh=_mesh,
    compiler_params=pltpu.CompilerParams(needs_layout_passes=False),
    scratch_types=[
        pltpu.VMEM((NP,), jnp.float32),
        pltpu.VMEM((K,), jnp.float32),
        [pltpu.VMEM((K,), jnp.int32)] * 3,
        [pltpu.VMEM((K,), jnp.int32)] * 3,
        [pltpu.VMEM((K,), jnp.int32)] * 3,
        [pltpu.VMEM((K,), jnp.float32)] * 3,
        [pltpu.VMEM((K,), jnp.float32)] * 3,
        [pltpu.VMEM((K, D), jnp.float32)] * 3,
        [pltpu.SemaphoreType.DMA] * 3,
        [pltpu.SemaphoreType.DMA] * 3,
        [pltpu.SemaphoreType.DMA] * 3,
        [pltpu.SemaphoreType.DMA] * 3,
        [pltpu.SemaphoreType.DMA] * 3,
        pltpu.VMEM_SHARED((NP, D), jnp.float32),
    ],
)
def _edge_phase(src_hbm, dst_hbm, asrc_hbm, adst_hbm, h_hbm, acc_out, den_out,
                den_v, ex_v, sidx, didx, dsct, av, bv, rows,
                isem, gsem, asem, bsem, ssem, acc_sp):
    c = lax.axis_index("c")
    s = lax.axis_index("s")
    wid = c * NS + s
    ebase = wid * EPW

    zeros16 = jnp.zeros((16,), jnp.float32)

    @pl.loop(0, NP // 16)
    def _zero_den(g):
        den_v[pl.ds(g * 16, 16)] = zeros16

    @pl.loop(0, RCP)
    def _zero_stage(k):
        for j in range(D // 16):
            rows[0][k, pl.ds(j * 16, 16)] = zeros16

    for i in range(RPT // RCP):
        pltpu.sync_copy(rows[0], acc_sp.at[pl.ds(s * RPT + i * RCP, RCP)])
    plsc.subcore_barrier()

    def issue_idx(j, b):
        base = ebase + j * K
        pltpu.async_copy(src_hbm.at[pl.ds(base, K)], sidx[b], isem[b])
        pltpu.async_copy(dst_hbm.at[pl.ds(base, K)], didx[b], isem[b])

    def wait_idx(b):
        pltpu.make_async_copy(src_hbm.at[pl.ds(0, K)], sidx[b], isem[b]).wait()
        pltpu.make_async_copy(dst_hbm.at[pl.ds(0, K)], didx[b], isem[b]).wait()

    def issue_gathers(b):
        pltpu.async_copy(h_hbm.at[sidx[b]], rows[b], gsem[b])
        pltpu.async_copy(asrc_hbm.at[sidx[b]], av[b], asem[b])
        pltpu.async_copy(adst_hbm.at[didx[b]], bv[b], bsem[b])

    def wait_gathers(b):
        pltpu.make_async_copy(h_hbm.at[sidx[b]], rows[b], gsem[b]).wait()
        pltpu.make_async_copy(asrc_hbm.at[sidx[b]], av[b], asem[b]).wait()
        pltpu.make_async_copy(adst_hbm.at[didx[b]], bv[b], bsem[b]).wait()

    def wait_scatter(b):
        pltpu.make_async_copy(rows[b], acc_sp.at[dsct[b]], ssem[b]).wait()

    issue_idx(0, 0)
    wait_idx(0)
    issue_gathers(0)
    issue_idx(1, 1)

    @pl.loop(0, NCHUNK + 1, step=3)
    def _chunks(j0):
        for u in range(3):
            j = j0 + u
            b = u % 3
            n = (u + 1) % 3
            p = (u + 2) % 3

            @pl.when(j < NCHUNK)
            def _():
                @pl.when(j > 1)
                def _():
                    wait_scatter(n)

                @pl.when(j + 1 < NCHUNK)
                def _():
                    wait_idx(n)
                    issue_gathers(n)

                @pl.when(j + 2 < NCHUNK)
                def _():
                    issue_idx(j + 2, p)

                wait_gathers(b)
                for g in range(K // 16):
                    sl = pl.ds(g * 16, 16)
                    d16 = didx[b][sl]
                    a = av[b][sl] + bv[b][sl]
                    ex = jnp.exp(jnp.maximum(a, 0.2 * a))
                    plsc.addupdate_scatter(den_v, [d16], ex)
                    ex_v[sl] = ex
                    dsct[b][sl] = d16

                @pl.loop(0, K, unroll=4)
                def _scale(k):
                    e16 = plsc.load_gather(ex_v, [jnp.full((16,), k, jnp.int32)])
                    for jj in range(D // 16):
                        sl = pl.ds(jj * 16, 16)
                        rows[b][k, sl] = rows[b][k, sl] * e16

                pltpu.async_copy(rows[b], acc_sp.at[dsct[b]], ssem[b],
                                 add=True)

    wait_scatter((NCHUNK - 2) % 3)
    wait_scatter((NCHUNK - 1) % 3)

    plsc.subcore_barrier()
    pltpu.sync_copy(den_v, den_out.at[wid])
    for i in range(RPT // RCP):
        r0 = s * RPT + i * RCP
        pltpu.sync_copy(acc_sp.at[pl.ds(r0, RCP)], rows[0])
        pltpu.sync_copy(rows[0], acc_out.at[c, pl.ds(r0, RCP)])


def _tc1_body(x_ref, w_ref, as_ref, ad_ref, h_ref, av_ref, bv_ref):
    h = jnp.dot(x_ref[...], w_ref[...], preferred_element_type=jnp.float32)
    h_ref[...] = h
    av_ref[...] = jnp.sum(h * as_ref[...], axis=1, keepdims=True)
    bv_ref[...] = jnp.sum(h * ad_ref[...], axis=1, keepdims=True)


def _den_col(den_ref):
    ones = jnp.ones((NW, 1), jnp.float32)
    col = lax.dot_general(den_ref[...], ones, (((0,), (0,)), ((), ())),
                          preferred_element_type=jnp.float32)
    return col[:N]


def _bn_relu(y, g_ref, be_ref):
    mean = jnp.mean(y, axis=0, keepdims=True)
    var = jnp.mean((y - mean) ** 2, axis=0, keepdims=True)
    return jnp.maximum(
        g_ref[...] * (y - mean) / jnp.sqrt(var + EPS_BN) + be_ref[...], 0.0)


def _tc2_body(acc_ref, den_ref, b_ref, g_ref, be_ref, w2_ref, as2_ref,
              ad2_ref, h2_ref, av_ref, bv_ref):
    a = acc_ref[...]
    y = (a[0, :N] + a[1, :N]) / (_den_col(den_ref) + 1e-16) + b_ref[...]
    z = _bn_relu(y, g_ref, be_ref)
    h2 = jnp.dot(z, w2_ref[...], preferred_element_type=jnp.float32)
    h2_ref[...] = h2
    av_ref[...] = jnp.sum(h2 * as2_ref[...], axis=1, keepdims=True)
    bv_ref[...] = jnp.sum(h2 * ad2_ref[...], axis=1, keepdims=True)


def _tc3_body(acc_ref, den_ref, batch_ref, b_ref, g_ref, be_ref, wfc_ref,
              bfc_ref, out_ref):
    a = acc_ref[...]
    y = (a[0, :N] + a[1, :N]) / (_den_col(den_ref) + 1e-16) + b_ref[...]
    z = _bn_relu(y, g_ref, be_ref)
    onehot = (batch_ref[...] == lax.broadcasted_iota(jnp.int32, (1, G), 1)
              ).astype(jnp.float32)
    sums = lax.dot_general(onehot, z, (((0,), (0,)), ((), ())),
                           preferred_element_type=jnp.float32)
    cnt = lax.dot_general(onehot, jnp.ones((N, 1), jnp.float32),
                          (((0,), (0,)), ((), ())),
                          preferred_element_type=jnp.float32)
    pool = sums / jnp.maximum(cnt, 1.0)
    out_ref[...] = jnp.dot(pool, wfc_ref[...],
                           preferred_element_type=jnp.float32) + bfc_ref[...]


_tc1 = pl.pallas_call(
    _tc1_body, out_shape=[_sds((N, D)), _sds((N, 1)), _sds((N, 1))])
_tc2 = pl.pallas_call(
    _tc2_body, out_shape=[_sds((N, D)), _sds((N, 1)), _sds((N, 1))])
_tc3 = pl.pallas_call(_tc3_body, out_shape=_sds((G, D_OUT)))


def kernel(x, edge_index, batch, W1, att_src1, att_dst1, b1, gamma1, beta1,
           W2, att_src2, att_dst2, b2, gamma2, beta2, Wfc, bfc):
    src = edge_index[0]
    dst = edge_index[1]
    h1, a1s, a1d = _tc1(x, W1, att_src1, att_dst1)
    acc1, den1 = _edge_phase(src, dst, a1s.reshape(N), a1d.reshape(N), h1)
    h2, a2s, a2d = _tc2(acc1, den1, b1.reshape(1, D), gamma1.reshape(1, D),
                        beta1.reshape(1, D), W2, att_src2, att_dst2)
    acc2, den2 = _edge_phase(src, dst, a2s.reshape(N), a2d.reshape(N), h2)
    return _tc3(acc2, den2, batch.reshape(N, 1), b2.reshape(1, D),
                gamma2.reshape(1, D), beta2.reshape(1, D), Wfc,
                bfc.reshape(1, D_OUT))


# speedup vs baseline: 52.7639x; 1.0156x over previous
import functools

import jax
import jax.numpy as jnp
from jax import lax
from jax.experimental import pallas as pl
from jax.experimental.pallas import tpu as pltpu
from jax.experimental.pallas import tpu_sc as plsc

N = 10000
E = 320000
D = 128
G = 128
D_OUT = 64
EPS_BN = 1e-5

NC = 2
NS = 16
NW = NC * NS
EPW = E // NW
K = 80
NCHUNK = EPW // K
NP = 10240
RPT = NP // NS
RCP = K


def _sds(shape, dtype=jnp.float32):
    return jax.ShapeDtypeStruct(shape, dtype)


_mesh = plsc.VectorSubcoreMesh(
    core_axis_name="c", subcore_axis_name="s", num_cores=NC, num_subcores=NS
)


@functools.partial(
    pl.kernel,
    out_type=[_sds((NC, NP, D)), _sds((NW, NP))],
    mesh=_mesh,
    compiler_params=pltpu.CompilerParams(needs_layout_passes=False),
    scratch_types=[
        pltpu.VMEM((NP,), jnp.float32),
        pltpu.VMEM((K,), jnp.float32),
        [pltpu.VMEM((K,), jnp.int32)] * 3,
        [pltpu.VMEM((K,), jnp.int32)] * 3,
        [pltpu.VMEM((K,), jnp.int32)] * 3,
        [pltpu.VMEM((K,), jnp.float32)] * 3,
        [pltpu.VMEM((K,), jnp.float32)] * 3,
        [pltpu.VMEM((K, D), jnp.float32)] * 3,
        [pltpu.SemaphoreType.DMA] * 3,
        [pltpu.SemaphoreType.DMA] * 3,
        [pltpu.SemaphoreType.DMA] * 3,
        [pltpu.SemaphoreType.DMA] * 3,
        [pltpu.SemaphoreType.DMA] * 3,
        pltpu.VMEM_SHARED((NP, D), jnp.float32),
    ],
)
def _edge_phase(src_hbm, dst_hbm, asrc_hbm, adst_hbm, h_hbm, acc_out, den_out,
                den_v, ex_v, sidx, didx, dsct, av, bv, rows,
                isem, gsem, asem, bsem, ssem, acc_sp):
    c = lax.axis_index("c")
    s = lax.axis_index("s")
    wid = c * NS + s
    ebase = wid * EPW

    zeros16 = jnp.zeros((16,), jnp.float32)

    @pl.loop(0, NP // 16)
    def _zero_den(g):
        den_v[pl.ds(g * 16, 16)] = zeros16

    @pl.loop(0, RCP)
    def _zero_stage(k):
        for j in range(D // 16):
            rows[0][k, pl.ds(j * 16, 16)] = zeros16

    for i in range(RPT // RCP):
        pltpu.async_copy(rows[0], acc_sp.at[pl.ds(s * RPT + i * RCP, RCP)],
                         gsem[0])
    for i in range(RPT // RCP):
        pltpu.make_async_copy(
            rows[0], acc_sp.at[pl.ds(s * RPT + i * RCP, RCP)], gsem[0]).wait()
    plsc.subcore_barrier()

    def issue_idx(j, b):
        base = ebase + j * K
        pltpu.async_copy(src_hbm.at[pl.ds(base, K)], sidx[b], isem[b])
        pltpu.async_copy(dst_hbm.at[pl.ds(base, K)], didx[b], isem[b])

    def wait_idx(b):
        pltpu.make_async_copy(src_hbm.at[pl.ds(0, K)], sidx[b], isem[b]).wait()
        pltpu.make_async_copy(dst_hbm.at[pl.ds(0, K)], didx[b], isem[b]).wait()

    def issue_gathers(b):
        pltpu.async_copy(h_hbm.at[sidx[b]], rows[b], gsem[b])
        pltpu.async_copy(asrc_hbm.at[sidx[b]], av[b], asem[b])
        pltpu.async_copy(adst_hbm.at[didx[b]], bv[b], bsem[b])

    def wait_gathers(b):
        pltpu.make_async_copy(h_hbm.at[sidx[b]], rows[b], gsem[b]).wait()
        pltpu.make_async_copy(asrc_hbm.at[sidx[b]], av[b], asem[b]).wait()
        pltpu.make_async_copy(adst_hbm.at[didx[b]], bv[b], bsem[b]).wait()

    def wait_scatter(b):
        pltpu.make_async_copy(rows[b], acc_sp.at[dsct[b]], ssem[b]).wait()

    issue_idx(0, 0)
    wait_idx(0)
    issue_gathers(0)
    issue_idx(1, 1)

    @pl.loop(0, NCHUNK + 1, step=3)
    def _chunks(j0):
        for u in range(3):
            j = j0 + u
            b = u % 3
            n = (u + 1) % 3
            p = (u + 2) % 3

            @pl.when(j < NCHUNK)
            def _():
                @pl.when(j > 1)
                def _():
                    wait_scatter(n)

                @pl.when(j + 1 < NCHUNK)
                def _():
                    wait_idx(n)
                    issue_gathers(n)

                @pl.when(j + 2 < NCHUNK)
                def _():
                    issue_idx(j + 2, p)

                wait_gathers(b)
                for g in range(K // 16):
                    sl = pl.ds(g * 16, 16)
                    d16 = didx[b][sl]
                    a = av[b][sl] + bv[b][sl]
                    ex = jnp.exp(jnp.maximum(a, 0.2 * a))
                    plsc.addupdate_scatter(den_v, [d16], ex)
                    ex_v[sl] = ex
                    dsct[b][sl] = d16

                @pl.loop(0, K, unroll=4)
                def _scale(k):
                    e16 = plsc.load_gather(ex_v, [jnp.full((16,), k, jnp.int32)])
                    for jj in range(D // 16):
                        sl = pl.ds(jj * 16, 16)
                        rows[b][k, sl] = rows[b][k, sl] * e16

                pltpu.async_copy(rows[b], acc_sp.at[dsct[b]], ssem[b],
                                 add=True)

    wait_scatter((NCHUNK - 2) % 3)
    wait_scatter((NCHUNK - 1) % 3)

    plsc.subcore_barrier()
    pltpu.async_copy(den_v, den_out.at[wid], isem[0])
    NPCS = RPT // RCP
    for i in range(NPCS):
        r0 = s * RPT + i * RCP
        b = i % 2
        if i >= 2:
            rp = s * RPT + (i - 2) * RCP
            pltpu.make_async_copy(
                rows[b], acc_out.at[c, pl.ds(rp, RCP)], gsem[b]).wait()
        pltpu.sync_copy(acc_sp.at[pl.ds(r0, RCP)], rows[b])
        pltpu.async_copy(rows[b], acc_out.at[c, pl.ds(r0, RCP)], gsem[b])
    for i in range(NPCS - 2, NPCS):
        r0 = s * RPT + i * RCP
        pltpu.make_async_copy(
            rows[i % 2], acc_out.at[c, pl.ds(r0, RCP)], gsem[i % 2]).wait()
    pltpu.make_async_copy(den_v, den_out.at[wid], isem[0]).wait()


def _tc1_body(x_ref, w_ref, as_ref, ad_ref, h_ref, av_ref, bv_ref):
    h = jnp.dot(x_ref[...], w_ref[...], preferred_element_type=jnp.float32)
    h_ref[...] = h
    av_ref[...] = jnp.sum(h * as_ref[...], axis=1, keepdims=True)
    bv_ref[...] = jnp.sum(h * ad_ref[...], axis=1, keepdims=True)


def _den_col(den_ref):
    ones = jnp.ones((NW, 1), jnp.float32)
    col = lax.dot_general(den_ref[...], ones, (((0,), (0,)), ((), ())),
                          preferred_element_type=jnp.float32)
    return col[:N]


def _bn_relu(y, g_ref, be_ref):
    mean = jnp.mean(y, axis=0, keepdims=True)
    var = jnp.mean((y - mean) ** 2, axis=0, keepdims=True)
    return jnp.maximum(
        g_ref[...] * (y - mean) / jnp.sqrt(var + EPS_BN) + be_ref[...], 0.0)


def _tc2_body(acc_ref, den_ref, b_ref, g_ref, be_ref, w2_ref, as2_ref,
              ad2_ref, h2_ref, av_ref, bv_ref):
    a = acc_ref[...]
    y = (a[0, :N] + a[1, :N]) / (_den_col(den_ref) + 1e-16) + b_ref[...]
    z = _bn_relu(y, g_ref, be_ref)
    h2 = jnp.dot(z, w2_ref[...], preferred_element_type=jnp.float32)
    h2_ref[...] = h2
    av_ref[...] = jnp.sum(h2 * as2_ref[...], axis=1, keepdims=True)
    bv_ref[...] = jnp.sum(h2 * ad2_ref[...], axis=1, keepdims=True)


def _tc3_body(acc_ref, den_ref, batch_ref, b_ref, g_ref, be_ref, wfc_ref,
              bfc_ref, out_ref):
    a = acc_ref[...]
    y = (a[0, :N] + a[1, :N]) / (_den_col(den_ref) + 1e-16) + b_ref[...]
    z = _bn_relu(y, g_ref, be_ref)
    onehot = (batch_ref[...] == lax.broadcasted_iota(jnp.int32, (1, G), 1)
              ).astype(jnp.float32)
    sums = lax.dot_general(onehot, z, (((0,), (0,)), ((), ())),
                           preferred_element_type=jnp.float32)
    cnt = lax.dot_general(onehot, jnp.ones((N, 1), jnp.float32),
                          (((0,), (0,)), ((), ())),
                          preferred_element_type=jnp.float32)
    pool = sums / jnp.maximum(cnt, 1.0)
    out_ref[...] = jnp.dot(pool, wfc_ref[...],
                           preferred_element_type=jnp.float32) + bfc_ref[...]


_tc1 = pl.pallas_call(
    _tc1_body, out_shape=[_sds((N, D)), _sds((N, 1)), _sds((N, 1))])
_tc2 = pl.pallas_call(
    _tc2_body, out_shape=[_sds((N, D)), _sds((N, 1)), _sds((N, 1))])
_tc3 = pl.pallas_call(_tc3_body, out_shape=_sds((G, D_OUT)))


def kernel(x, edge_index, batch, W1, att_src1, att_dst1, b1, gamma1, beta1,
           W2, att_src2, att_dst2, b2, gamma2, beta2, Wfc, bfc):
    src = edge_index[0]
    dst = edge_index[1]
    h1, a1s, a1d = _tc1(x, W1, att_src1, att_dst1)
    acc1, den1 = _edge_phase(src, dst, a1s.reshape(N), a1d.reshape(N), h1)
    h2, a2s, a2d = _tc2(acc1, den1, b1.reshape(1, D), gamma1.reshape(1, D),
                        beta1.reshape(1, D), W2, att_src2, att_dst2)
    acc2, den2 = _edge_phase(src, dst, a2s.reshape(N), a2d.reshape(N), h2)
    return _tc3(acc2, den2, batch.reshape(N, 1), b2.reshape(1, D),
                gamma2.reshape(1, D), beta2.reshape(1, D), Wfc,
                bfc.reshape(1, D_OUT))
